# baseline (speedup 1.0000x reference)
import functools

import jax
import jax.numpy as jnp
from jax import lax
from jax.experimental import pallas as pl
from jax.experimental.pallas import tpu as pltpu
from jax.experimental.pallas import tpu_sc as plsc

C = 64
K = 16
DEPTH = 4
N_TOKENS = 4096
IN_FEATURES = 1024
OUT_FEATURES = 1024

BN = 1024
NW = 32
NH = N_TOKENS // 2
NPT = NH // NW
TG = 16

_LVL = [0, 1, 1, 2, 2, 2, 2, 3, 3, 3, 3, 3, 3, 3, 3]


def _descend(b):
    d0 = b[0]
    d1 = d0 * b[2] + (1.0 - d0) * b[1]
    d2l = d1 * b[4] + (1.0 - d1) * b[3]
    d2r = d1 * b[6] + (1.0 - d1) * b[5]
    d2 = d0 * d2r + (1.0 - d0) * d2l
    d3ll = d2 * b[8] + (1.0 - d2) * b[7]
    d3lr = d2 * b[10] + (1.0 - d2) * b[9]
    d3rl = d2 * b[12] + (1.0 - d2) * b[11]
    d3rr = d2 * b[14] + (1.0 - d2) * b[13]
    d3l = d1 * d3lr + (1.0 - d1) * d3ll
    d3r = d1 * d3rr + (1.0 - d1) * d3rl
    d3 = d0 * d3r + (1.0 - d0) * d3l
    return d0 * 8.0 + d1 * 4.0 + d2 * 2.0 + d3


def _make_sc_body(row0):
  def _sc_encode_body(x_hbm, dims_hbm, thr_hbm, idx_hbm, dims_v, thr_v,
                      xbuf0, xbuf1, obuf, sem0, sem1):
    wid = lax.axis_index("s") * 2 + lax.axis_index("c")
    base = wid * NPT
    pltpu.sync_copy(dims_hbm, dims_v)
    pltpu.sync_copy(thr_hbm, thr_v)

    def src(g):
        return x_hbm.at[pl.ds(row0 + base + g * TG, TG), :]

    def process(buf, g):
        for q in range(4):
            cols = [dims_v[pl.ds((l * 4 + q) * 16, 16)] for l in range(4)]
            thr = [thr_v[pl.ds(j * C + q * 16, 16)] for j in range(15)]

            def token(t, carry2, _cols=cols, _thr=thr):
                rows = jnp.full((16,), t, dtype=jnp.int32)
                vals = [plsc.load_gather(buf, [rows, _cols[l]])
                        for l in range(4)]
                b = [jnp.where(vals[_LVL[j]] > _thr[j], 1.0, 0.0)
                     for j in range(15)]
                obuf[g * TG + t, pl.ds(q * 16, 16)] = _descend(b)
                return carry2

            lax.fori_loop(0, TG, token, 0, unroll=False)

    ng = NPT // TG
    pltpu.async_copy(src(0), xbuf0, sem0)

    def gpair(p, carry):
        g0 = 2 * p
        pltpu.async_copy(src(g0 + 1), xbuf1, sem1)
        pltpu.make_async_copy(src(g0), xbuf0, sem0).wait()
        process(xbuf0, g0)

        @pl.when(g0 + 2 < ng)
        def _():
            pltpu.async_copy(src(g0 + 2), xbuf0, sem0)

        pltpu.make_async_copy(src(g0 + 1), xbuf1, sem1).wait()
        process(xbuf1, g0 + 1)
        return carry

    lax.fori_loop(0, ng // 2, gpair, 0, unroll=False)
    pltpu.sync_copy(obuf, idx_hbm.at[pl.ds(base, NPT), :])

  return _sc_encode_body


def _sc_encode(x, dims_lm, thr_jm, row0):
    kern = functools.partial(
        pl.kernel,
        mesh=plsc.VectorSubcoreMesh(core_axis_name="c", subcore_axis_name="s"),
        out_type=jax.ShapeDtypeStruct((NH, C), jnp.float32),
        scratch_types=[
            pltpu.VMEM((C * DEPTH,), jnp.int32),
            pltpu.VMEM((15 * C,), jnp.float32),
            pltpu.VMEM((TG, IN_FEATURES), jnp.float32),
            pltpu.VMEM((TG, IN_FEATURES), jnp.float32),
            pltpu.VMEM((NPT, C), jnp.float32),
            pltpu.SemaphoreType.DMA,
            pltpu.SemaphoreType.DMA,
        ],
        compiler_params=pltpu.CompilerParams(needs_layout_passes=False),
    )(_make_sc_body(row0))
    return kern(x, dims_lm, thr_jm)


def _tc_kernel(idx_ref, lut_ref, out_ref):
    _tc_compute(idx_ref, lut_ref, out_ref)


def _tc_kernel_alias(idx_ref, lut_ref, alias_ref, out_ref):
    del alias_ref
    _tc_compute(idx_ref, lut_ref, out_ref)


def _tc_compute(idx_ref, lut_ref, out_ref):
    idxT = idx_ref[...].T
    encT = jnp.concatenate(
        [jnp.where(idxT == float(k), 1.0, 0.0) for k in range(K)], axis=0)
    out_ref[...] = lax.dot_general(
        encT.astype(jnp.bfloat16), lut_ref[...],
        dimension_numbers=(((0,), (0,)), ((), ())),
        preferred_element_type=jnp.float32)


@jax.jit
def _run(x, dims, thresholds, lut):
    dims_lm = dims.astype(jnp.int32).reshape(C, DEPTH).T.reshape(C * DEPTH)
    thr_jm = thresholds.reshape(C, K - 1).T.reshape((K - 1) * C)
    bits = lax.bitcast_convert_type(thr_jm, jnp.int32)
    low = bits & jnp.int32(0xFFFF)
    hi = bits & jnp.int32(-65536)
    neg = bits < 0
    off_grid = low != 0
    b_lo = jnp.where(off_grid & neg, hi + 0x10000, hi)
    b_next = jnp.where(neg, jnp.where(off_grid, hi, hi - 0x10000),
                       hi + 0x10000)
    thr_m = 0.5 * (lax.bitcast_convert_type(b_lo, jnp.float32)
                   + lax.bitcast_convert_type(b_next, jnp.float32))
    lutK = jnp.transpose(lut, (2, 1, 0)).reshape(K * C, OUT_FEATURES)
    lutK = lutK.astype(jnp.bfloat16)

    idx1 = _sc_encode(x, dims_lm, thr_m, 0)
    idx2 = _sc_encode(x, dims_lm, thr_m, NH)
    nb_h = NH // BN
    out1 = pl.pallas_call(
        _tc_kernel,
        grid=(nb_h,),
        in_specs=[
            pl.BlockSpec((BN, C), lambda i: (i, 0)),
            pl.BlockSpec((K * C, OUT_FEATURES), lambda i: (0, 0)),
        ],
        out_specs=pl.BlockSpec((BN, OUT_FEATURES), lambda i: (i, 0)),
        out_shape=jax.ShapeDtypeStruct((N_TOKENS, OUT_FEATURES), jnp.float32),
    )(idx1, lutK)
    return pl.pallas_call(
        _tc_kernel_alias,
        grid=(nb_h,),
        in_specs=[
            pl.BlockSpec((BN, C), lambda i: (i, 0)),
            pl.BlockSpec((K * C, OUT_FEATURES), lambda i: (0, 0)),
            pl.BlockSpec(memory_space=pltpu.MemorySpace.HBM),
        ],
        out_specs=pl.BlockSpec((BN, OUT_FEATURES), lambda i: (i + nb_h, 0)),
        out_shape=jax.ShapeDtypeStruct((N_TOKENS, OUT_FEATURES), jnp.float32),
        input_output_aliases={2: 0},
    )(idx2, lutK, out1)


def kernel(x, dims, thresholds, lut, S, T):
    return _run(x, dims, thresholds, lut)


# speedup vs baseline: 1.1545x; 1.1545x over previous
import functools

import jax
import jax.numpy as jnp
from jax import lax
from jax.experimental import pallas as pl
from jax.experimental.pallas import tpu as pltpu
from jax.experimental.pallas import tpu_sc as plsc

C = 64
K = 16
DEPTH = 4
N_TOKENS = 4096
IN_FEATURES = 1024
OUT_FEATURES = 1024

BN = 1024
NW = 32
NPT = N_TOKENS // NW
TG = 16

_LVL = [0, 1, 1, 2, 2, 2, 2, 3, 3, 3, 3, 3, 3, 3, 3]


def _descend(b):
    d0 = b[0]
    d1 = d0 * b[2] + (1.0 - d0) * b[1]
    d2l = d1 * b[4] + (1.0 - d1) * b[3]
    d2r = d1 * b[6] + (1.0 - d1) * b[5]
    d2 = d0 * d2r + (1.0 - d0) * d2l
    d3ll = d2 * b[8] + (1.0 - d2) * b[7]
    d3lr = d2 * b[10] + (1.0 - d2) * b[9]
    d3rl = d2 * b[12] + (1.0 - d2) * b[11]
    d3rr = d2 * b[14] + (1.0 - d2) * b[13]
    d3l = d1 * d3lr + (1.0 - d1) * d3ll
    d3r = d1 * d3rr + (1.0 - d1) * d3rl
    d3 = d0 * d3r + (1.0 - d0) * d3l
    return d0 * 8.0 + d1 * 4.0 + d2 * 2.0 + d3


def _sc_encode_body(x_hbm, dims_hbm, thr_hbm, idx_hbm, dims_v, thr_v,
                    xbuf0, xbuf1, obuf, sem0, sem1):
    wid = lax.axis_index("s") * 2 + lax.axis_index("c")
    base = wid * NPT
    pltpu.sync_copy(dims_hbm, dims_v)
    pltpu.sync_copy(thr_hbm, thr_v)

    def src(g):
        return x_hbm.at[pl.ds(base + g * TG, TG), :]

    def process(buf, g):
        for q in range(4):
            cols = [dims_v[pl.ds((l * 4 + q) * 16, 16)] for l in range(4)]
            thr = [thr_v[pl.ds(j * C + q * 16, 16)] for j in range(15)]

            def token(t, carry2, _cols=cols, _thr=thr):
                rows = jnp.full((16,), t, dtype=jnp.int32)
                vals = [plsc.load_gather(buf, [rows, _cols[l]])
                        for l in range(4)]
                b = [jnp.where(vals[_LVL[j]] > _thr[j], 1.0, 0.0)
                     for j in range(15)]
                obuf[g * TG + t, pl.ds(q * 16, 16)] = _descend(b)
                return carry2

            lax.fori_loop(0, TG, token, 0, unroll=False)

    ng = NPT // TG
    pltpu.async_copy(src(0), xbuf0, sem0)

    def gpair(p, carry):
        g0 = 2 * p
        pltpu.async_copy(src(g0 + 1), xbuf1, sem1)
        pltpu.make_async_copy(src(g0), xbuf0, sem0).wait()
        process(xbuf0, g0)

        @pl.when(g0 + 2 < ng)
        def _():
            pltpu.async_copy(src(g0 + 2), xbuf0, sem0)

        pltpu.make_async_copy(src(g0 + 1), xbuf1, sem1).wait()
        process(xbuf1, g0 + 1)
        return carry

    lax.fori_loop(0, ng // 2, gpair, 0, unroll=False)
    pltpu.sync_copy(obuf, idx_hbm.at[pl.ds(base, NPT), :])


def _sc_encode(x, dims_lm, thr_jm):
    kern = functools.partial(
        pl.kernel,
        mesh=plsc.VectorSubcoreMesh(core_axis_name="c", subcore_axis_name="s"),
        out_type=jax.ShapeDtypeStruct((N_TOKENS, C), jnp.float32),
        scratch_types=[
            pltpu.VMEM((C * DEPTH,), jnp.int32),
            pltpu.VMEM((15 * C,), jnp.float32),
            pltpu.VMEM((TG, IN_FEATURES), jnp.float32),
            pltpu.VMEM((TG, IN_FEATURES), jnp.float32),
            pltpu.VMEM((NPT, C), jnp.float32),
            pltpu.SemaphoreType.DMA,
            pltpu.SemaphoreType.DMA,
        ],
        compiler_params=pltpu.CompilerParams(needs_layout_passes=False),
    )(_sc_encode_body)
    return kern(x, dims_lm, thr_jm)


def _tc_kernel(idx_ref, lut_ref, out_ref):
    idxT = idx_ref[...].T
    encT = jnp.concatenate(
        [jnp.where(idxT == float(k), 1.0, 0.0) for k in range(K)], axis=0)
    out_ref[...] = lax.dot_general(
        encT.astype(jnp.bfloat16), lut_ref[...],
        dimension_numbers=(((0,), (0,)), ((), ())),
        preferred_element_type=jnp.float32)


@jax.jit
def _run(x, dims, thresholds, lut):
    dims_lm = dims.astype(jnp.int32).reshape(C, DEPTH).T.reshape(C * DEPTH)
    thr_jm = thresholds.reshape(C, K - 1).T.reshape((K - 1) * C)
    bits = lax.bitcast_convert_type(thr_jm, jnp.int32)
    low = bits & jnp.int32(0xFFFF)
    hi = bits & jnp.int32(-65536)
    neg = bits < 0
    off_grid = low != 0
    b_lo = jnp.where(off_grid & neg, hi + 0x10000, hi)
    b_next = jnp.where(neg, jnp.where(off_grid, hi, hi - 0x10000),
                       hi + 0x10000)
    thr_m = 0.5 * (lax.bitcast_convert_type(b_lo, jnp.float32)
                   + lax.bitcast_convert_type(b_next, jnp.float32))
    lutK = jnp.transpose(lut, (2, 1, 0)).reshape(K * C, OUT_FEATURES)
    lutK = lutK.astype(jnp.bfloat16)

    idx = _sc_encode(x, dims_lm, thr_m)

    return pl.pallas_call(
        _tc_kernel,
        grid=(N_TOKENS // BN,),
        in_specs=[
            pl.BlockSpec((BN, C), lambda i: (i, 0)),
            pl.BlockSpec((K * C, OUT_FEATURES), lambda i: (0, 0)),
        ],
        out_specs=pl.BlockSpec((BN, OUT_FEATURES), lambda i: (i, 0)),
        out_shape=jax.ShapeDtypeStruct((N_TOKENS, OUT_FEATURES), jnp.float32),
    )(idx, lutK)


def kernel(x, dims, thresholds, lut, S, T):
    return _run(x, dims, thresholds, lut)
